# Initial kernel scaffold; baseline (speedup 1.0000x reference)
import jax
import jax.numpy as jnp
from jax.experimental import pallas as pl


def kernel(node_list, edge_list, edge_att, W_l, b_l, W_r, b_r, att, W_e, bias, lin_W, lin_b):
    raise NotImplementedError("write your pallas kernel here")


# speedup vs baseline: 4.7045x; 4.7045x over previous
import functools

import jax
import jax.numpy as jnp
from jax import lax
from jax.experimental import pallas as pl
from jax.experimental.pallas import tpu as pltpu
from jax.experimental.pallas import tpu_sc as plsc

N = 10000
E = 320000
CH = 128
NC = 2
NS = 16
NW = NC * NS
EPT = E // NW
B = 80
NCHUNK = EPT // B
RPT = N // NS
NP = 10240
RPT2 = NP // NS
WCH = 32
NEG = -3.0e38

_mesh = plsc.VectorSubcoreMesh(core_axis_name="c", subcore_axis_name="s")


def _proj_body(x_ref, wlt_ref, wrt_ref, bl_ref, br_ref, xl_ref, xr_ref):
    x = x_ref[...]
    xl_ref[...] = jnp.dot(x, wlt_ref[...], preferred_element_type=jnp.float32) + bl_ref[...]
    xr_ref[...] = jnp.dot(x, wrt_ref[...], preferred_element_type=jnp.float32) + br_ref[...]


def _proj(x, wlt, wrt, bl, br):
    nb = 1000
    return pl.pallas_call(
        _proj_body,
        grid=(N // nb,),
        in_specs=[
            pl.BlockSpec((nb, CH), lambda i: (i, 0)),
            pl.BlockSpec((CH, CH), lambda i: (0, 0)),
            pl.BlockSpec((CH, CH), lambda i: (0, 0)),
            pl.BlockSpec((1, CH), lambda i: (0, 0)),
            pl.BlockSpec((1, CH), lambda i: (0, 0)),
        ],
        out_specs=[
            pl.BlockSpec((nb, CH), lambda i: (i, 0)),
            pl.BlockSpec((nb, CH), lambda i: (i, 0)),
        ],
        out_shape=[
            jax.ShapeDtypeStruct((N, CH), jnp.float32),
            jax.ShapeDtypeStruct((N, CH), jnp.float32),
        ],
    )(x, wlt, wrt, bl, br)


@functools.partial(
    pl.kernel,
    out_type=(
        jax.ShapeDtypeStruct((E, CH), jnp.float32),
        jax.ShapeDtypeStruct((E, CH), jnp.float32),
    ),
    mesh=_mesh,
    compiler_params=pltpu.CompilerParams(needs_layout_passes=False),
    scratch_types=[
        pltpu.VMEM((B,), jnp.int32),
        pltpu.VMEM((B,), jnp.int32),
        pltpu.VMEM((B, CH), jnp.float32),
        pltpu.VMEM((B, CH), jnp.float32),
        pltpu.SemaphoreType.DMA,
        pltpu.SemaphoreType.DMA,
    ],
)
def _gather_kernel(xl_hbm, xr_hbm, src_hbm, dst_hbm,
                   gl_hbm, gr_hbm,
                   src_v, dst_v, rows_l, rows_r, sem1, sem2):
    wid = lax.axis_index("s") * NC + lax.axis_index("c")
    base = wid * EPT

    def chunk_body(k, _):
        off = base + k * B
        pltpu.sync_copy(src_hbm.at[pl.ds(off, B)], src_v)
        pltpu.sync_copy(dst_hbm.at[pl.ds(off, B)], dst_v)
        cl = pltpu.async_copy(xl_hbm.at[src_v], rows_l, sem1)
        cr = pltpu.async_copy(xr_hbm.at[dst_v], rows_r, sem2)
        cl.wait()
        cr.wait()
        pltpu.sync_copy(rows_l, gl_hbm.at[pl.ds(off, B)])
        pltpu.sync_copy(rows_r, gr_hbm.at[pl.ds(off, B)])
        return 0

    lax.fori_loop(0, NCHUNK, chunk_body, 0)


RB = 4000
NRB = E // RB


def _alpha_body(gl_ref, gr_ref, ea_ref, we_ref, att_ref, a_ref, g_ref, acc):
    i = pl.program_id(0)
    m = gl_ref[...] + gr_ref[...] + ea_ref[...] * we_ref[...]
    m = jnp.maximum(m, 0.2 * m)
    a = jnp.dot(m, att_ref[...], preferred_element_type=jnp.float32)
    a_ref[...] = a

    @pl.when(i == 0)
    def _():
        acc[...] = jnp.full_like(acc, NEG)

    acc[...] = jnp.maximum(acc[...], jnp.max(a, axis=0, keepdims=True))

    @pl.when(i == NRB - 1)
    def _():
        g_ref[...] = acc[...]


def _alpha(gl, gr, ea, we, att2):
    return pl.pallas_call(
        _alpha_body,
        grid=(NRB,),
        in_specs=[
            pl.BlockSpec((RB, CH), lambda i: (i, 0)),
            pl.BlockSpec((RB, CH), lambda i: (i, 0)),
            pl.BlockSpec((RB, 1), lambda i: (i, 0)),
            pl.BlockSpec((1, CH), lambda i: (0, 0)),
            pl.BlockSpec((CH, 1), lambda i: (0, 0)),
        ],
        out_specs=[
            pl.BlockSpec((RB, 1), lambda i: (i, 0)),
            pl.BlockSpec((1, 1), lambda i: (0, 0)),
        ],
        out_shape=[
            jax.ShapeDtypeStruct((E, 1), jnp.float32),
            jax.ShapeDtypeStruct((1, 1), jnp.float32),
        ],
        scratch_shapes=[pltpu.VMEM((1, 1), jnp.float32)],
    )(gl, gr, ea, we, att2)


def _exp_body(a_ref, g_ref, e_ref):
    e_ref[...] = jnp.exp(a_ref[...] - g_ref[0, 0])


def _expk(alpha, gmax):
    return pl.pallas_call(
        _exp_body,
        grid=(NRB,),
        in_specs=[
            pl.BlockSpec((RB, 1), lambda i: (i, 0)),
            pl.BlockSpec((1, 1), lambda i: (0, 0)),
        ],
        out_specs=pl.BlockSpec((RB, 1), lambda i: (i, 0)),
        out_shape=jax.ShapeDtypeStruct((E, 1), jnp.float32),
    )(alpha, gmax)


@functools.partial(
    pl.kernel,
    out_type=jax.ShapeDtypeStruct((NW * N,), jnp.float32),
    mesh=_mesh,
    compiler_params=pltpu.CompilerParams(needs_layout_passes=False),
    scratch_types=[
        pltpu.VMEM((B,), jnp.int32),
        pltpu.VMEM((B,), jnp.float32),
        pltpu.VMEM((N,), jnp.float32),
    ],
)
def _ssum_kernel(dst_hbm, ex_hbm, ssum_p_hbm, dst_v, ex_v, ssum_v):
    wid = lax.axis_index("s") * NC + lax.axis_index("c")
    base = wid * EPT

    def zbody(i, _):
        ssum_v[pl.ds(i * 16, 16)] = jnp.zeros((16,), jnp.float32)
        return 0

    lax.fori_loop(0, N // 16, zbody, 0)

    def chunk_body(k, _):
        off = base + k * B
        pltpu.sync_copy(dst_hbm.at[pl.ds(off, B)], dst_v)
        pltpu.sync_copy(ex_hbm.at[pl.ds(off, B)], ex_v)
        for q in range(B // 16):
            dv = dst_v[pl.ds(16 * q, 16)]
            ev = ex_v[pl.ds(16 * q, 16)]
            plsc.addupdate_scatter(ssum_v, [dv], ev)
        return 0

    lax.fori_loop(0, NCHUNK, chunk_body, 0)
    pltpu.sync_copy(ssum_v, ssum_p_hbm.at[pl.ds(wid * N, N)])


def _rsum_body(p_ref, o_ref):
    o_ref[...] = jnp.sum(p_ref[...], axis=0, keepdims=True)


def _rsum(p):
    return pl.pallas_call(
        _rsum_body,
        out_shape=jax.ShapeDtypeStruct((1, N), jnp.float32),
    )(p)


@functools.partial(
    pl.kernel,
    out_type=jax.ShapeDtypeStruct((E,), jnp.float32),
    mesh=_mesh,
    compiler_params=pltpu.CompilerParams(needs_layout_passes=False),
    scratch_types=[
        pltpu.VMEM((B,), jnp.int32),
        pltpu.VMEM((B,), jnp.float32),
        pltpu.VMEM((B,), jnp.float32),
        pltpu.VMEM((N,), jnp.float32),
    ],
)
def _norm_kernel(dst_hbm, ex_hbm, ssum_hbm, alphan_hbm,
                 dst_v, ex_v, an_v, ssum_v):
    wid = lax.axis_index("s") * NC + lax.axis_index("c")
    base = wid * EPT
    pltpu.sync_copy(ssum_hbm, ssum_v)

    def chunk_body(k, _):
        off = base + k * B
        pltpu.sync_copy(dst_hbm.at[pl.ds(off, B)], dst_v)
        pltpu.sync_copy(ex_hbm.at[pl.ds(off, B)], ex_v)
        for q in range(B // 16):
            dv = dst_v[pl.ds(16 * q, 16)]
            sv = plsc.load_gather(ssum_v, [dv])
            an_v[pl.ds(16 * q, 16)] = ex_v[pl.ds(16 * q, 16)] / (sv + 1e-16)
        pltpu.sync_copy(an_v, alphan_hbm.at[pl.ds(off, B)])
        return 0

    lax.fori_loop(0, NCHUNK, chunk_body, 0)


def _scale_body(an_ref, gl_ref, m_ref):
    m_ref[...] = an_ref[...] * gl_ref[...]


def _scale(alpha_n, gl):
    return pl.pallas_call(
        _scale_body,
        grid=(NRB,),
        in_specs=[
            pl.BlockSpec((RB, 1), lambda i: (i, 0)),
            pl.BlockSpec((RB, CH), lambda i: (i, 0)),
        ],
        out_specs=pl.BlockSpec((RB, CH), lambda i: (i, 0)),
        out_shape=jax.ShapeDtypeStruct((E, CH), jnp.float32),
    )(alpha_n, gl)


@functools.partial(
    pl.kernel,
    out_type=jax.ShapeDtypeStruct((NC, NP, CH), jnp.float32),
    mesh=_mesh,
    compiler_params=pltpu.CompilerParams(needs_layout_passes=False),
    scratch_types=[
        pltpu.VMEM((B,), jnp.int32),
        pltpu.VMEM((B, CH), jnp.float32),
        pltpu.VMEM((WCH, CH), jnp.float32),
        pltpu.VMEM_SHARED((NP, CH), jnp.float32),
    ],
)
def _agg_kernel(msg_hbm, dst_hbm, outp_hbm,
                dst_v, rows, stage, out_sh):
    cid = lax.axis_index("c")
    sid = lax.axis_index("s")
    wid = sid * NC + cid
    base = wid * EPT

    def zb(i, _):
        for j in range(8):
            stage[i, pl.ds(16 * j, 16)] = jnp.zeros((16,), jnp.float32)
        return 0

    lax.fori_loop(0, WCH, zb, 0)
    row0 = sid * RPT2

    def zs(t, _):
        pltpu.sync_copy(stage, out_sh.at[pl.ds(row0 + t * WCH, WCH)])
        return 0

    lax.fori_loop(0, RPT2 // WCH, zs, 0)
    plsc.subcore_barrier()

    def chunk_body(k, _):
        off = base + k * B
        pltpu.sync_copy(dst_hbm.at[pl.ds(off, B)], dst_v)
        pltpu.sync_copy(msg_hbm.at[pl.ds(off, B)], rows)
        pltpu.sync_copy(rows, out_sh.at[dst_v], add=True)
        return 0

    lax.fori_loop(0, NCHUNK, chunk_body, 0)
    plsc.subcore_barrier()

    def wb(t, _):
        pltpu.sync_copy(out_sh.at[pl.ds(row0 + t * WCH, WCH)], stage)
        pltpu.sync_copy(stage, outp_hbm.at[cid, pl.ds(row0 + t * WCH, WCH)])
        return 0

    lax.fori_loop(0, RPT2 // WCH, wb, 0)


FB = 1000
NFB = N // FB


def _final_body(p_ref, bias_ref, linwt_ref, linb_ref, o_ref, acc):
    i = pl.program_id(0)

    @pl.when(i == 0)
    def _():
        acc[...] = jnp.zeros_like(acc)

    h = jnp.maximum(p_ref[0] + p_ref[1] + bias_ref[...], 0.0)
    acc[...] += jnp.sum(h, axis=0, keepdims=True)
    pooled = acc[...] * jnp.float32(1.0 / N)
    logits = jnp.dot(pooled, linwt_ref[...], preferred_element_type=jnp.float32) \
        + linb_ref[...]
    m = jnp.max(logits, axis=1, keepdims=True)
    e = jnp.exp(logits - m)
    o_ref[...] = e / jnp.sum(e, axis=1, keepdims=True)


def _final(p, bias, linwt, linb):
    return pl.pallas_call(
        _final_body,
        grid=(NFB,),
        in_specs=[
            pl.BlockSpec((NC, FB, CH), lambda i: (0, i, 0)),
            pl.BlockSpec((1, CH), lambda i: (0, 0)),
            pl.BlockSpec((CH, 2), lambda i: (0, 0)),
            pl.BlockSpec((1, 2), lambda i: (0, 0)),
        ],
        out_specs=pl.BlockSpec((1, 2), lambda i: (0, 0)),
        out_shape=jax.ShapeDtypeStruct((1, 2), jnp.float32),
        scratch_shapes=[pltpu.VMEM((1, CH), jnp.float32)],
    )(p, bias, linwt, linb)


@jax.jit
def _run(node_list, edge_list, edge_att, W_l, b_l, W_r, b_r, att, W_e,
         bias, lin_W, lin_b):
    x = node_list[0].astype(jnp.float32)
    src = edge_list[0, 0].astype(jnp.int32)
    dst = edge_list[0, 1].astype(jnp.int32)
    ea = edge_att[0].astype(jnp.float32)
    xl, xr = _proj(x, W_l.T, W_r.T, b_l[None, :], b_r[None, :])
    gl, gr = _gather_kernel(xl, xr, src, dst)
    alpha, gmax = _alpha(gl, gr, ea, W_e.T, att[:, None])
    ex = _expk(alpha, gmax)
    ex1 = ex.reshape(-1)
    ssum_p = _ssum_kernel(dst, ex1)
    ssum = _rsum(ssum_p.reshape(NW, N)).reshape(-1)
    alpha_n = _norm_kernel(dst, ex1, ssum)
    msg = _scale(alpha_n[:, None], gl)
    outp = _agg_kernel(msg, dst)
    pred = _final(outp, bias[None, :], lin_W.T, lin_b[None, :])
    return pred, alpha_n


def kernel(node_list, edge_list, edge_att, W_l, b_l, W_r, b_r, att, W_e,
           bias, lin_W, lin_b):
    return _run(node_list, edge_list, edge_att, W_l, b_l, W_r, b_r, att,
                W_e, bias, lin_W, lin_b)


# speedup vs baseline: 6.7853x; 1.4423x over previous
import functools

import jax
import jax.numpy as jnp
from jax import lax
from jax.experimental import pallas as pl
from jax.experimental.pallas import tpu as pltpu
from jax.experimental.pallas import tpu_sc as plsc

N = 10000
E = 320000
CH = 128
NC = 2
NS = 16
NW = NC * NS
EPT = E // NW
B = 80
NCHUNK = EPT // B
NP = 10240
RPT2 = NP // NS
WCH = 32
NEG = -3.0e38

_mesh = plsc.VectorSubcoreMesh(core_axis_name="c", subcore_axis_name="s")
_sc_params = pltpu.CompilerParams(needs_layout_passes=False)


def _proj_body(x_ref, wlt_ref, wrt_ref, bl_ref, br_ref, xl_ref, xr_ref):
    x = x_ref[...]
    xl_ref[...] = jnp.dot(x, wlt_ref[...], preferred_element_type=jnp.float32) + bl_ref[...]
    xr_ref[...] = jnp.dot(x, wrt_ref[...], preferred_element_type=jnp.float32) + br_ref[...]


def _proj(x, wlt, wrt, bl, br):
    nb = 1000
    return pl.pallas_call(
        _proj_body,
        grid=(N // nb,),
        in_specs=[
            pl.BlockSpec((nb, CH), lambda i: (i, 0)),
            pl.BlockSpec((CH, CH), lambda i: (0, 0)),
            pl.BlockSpec((CH, CH), lambda i: (0, 0)),
            pl.BlockSpec((1, CH), lambda i: (0, 0)),
            pl.BlockSpec((1, CH), lambda i: (0, 0)),
        ],
        out_specs=[
            pl.BlockSpec((nb, CH), lambda i: (i, 0)),
            pl.BlockSpec((nb, CH), lambda i: (i, 0)),
        ],
        out_shape=[
            jax.ShapeDtypeStruct((N, CH), jnp.float32),
            jax.ShapeDtypeStruct((N, CH), jnp.float32),
        ],
    )(x, wlt, wrt, bl, br)


@functools.partial(
    pl.kernel,
    out_type=jax.ShapeDtypeStruct((E, CH), jnp.float32),
    mesh=_mesh,
    compiler_params=_sc_params,
    scratch_types=[
        pltpu.VMEM((NCHUNK, B), jnp.int32),
        pltpu.VMEM((NCHUNK, B), jnp.int32),
        pltpu.VMEM((B, CH), jnp.float32),
        pltpu.VMEM((B, CH), jnp.float32),
        pltpu.SemaphoreType.DMA,
        pltpu.SemaphoreType.DMA,
    ],
)
def _gs_kernel(xl_hbm, xr_hbm, src3_hbm, dst3_hbm, s_hbm,
               src_a, dst_a, rows_l, rows_r, sem1, sem2):
    wid = lax.axis_index("s") * NC + lax.axis_index("c")
    base = wid * EPT
    pltpu.sync_copy(src3_hbm.at[wid], src_a)
    pltpu.sync_copy(dst3_hbm.at[wid], dst_a)

    def chunk_body(k, _):
        cl = pltpu.async_copy(xl_hbm.at[src_a.at[k]], rows_l, sem1)
        cr = pltpu.async_copy(xr_hbm.at[dst_a.at[k]], rows_r, sem2)
        cl.wait()
        cr.wait()

        def addrow(i, _):
            for j in range(8):
                rows_l[i, pl.ds(16 * j, 16)] = rows_l[i, pl.ds(16 * j, 16)] \
                    + rows_r[i, pl.ds(16 * j, 16)]
            return 0

        lax.fori_loop(0, B, addrow, 0)
        pltpu.sync_copy(rows_l, s_hbm.at[pl.ds(base + k * B, B)])
        return 0

    lax.fori_loop(0, NCHUNK, chunk_body, 0)


RB = 4000
NRB = E // RB


def _alpha_body(s_ref, ea_ref, we_ref, att_ref, a_ref, g_ref, acc):
    i = pl.program_id(0)
    m = s_ref[...] + ea_ref[...] * we_ref[...]
    m = jnp.maximum(m, 0.2 * m)
    a = jnp.dot(m, att_ref[...], preferred_element_type=jnp.float32)
    a_ref[...] = a

    @pl.when(i == 0)
    def _():
        acc[...] = jnp.full_like(acc, NEG)

    acc[...] = jnp.maximum(acc[...], jnp.max(a, axis=0, keepdims=True))

    @pl.when(i == NRB - 1)
    def _():
        g_ref[...] = acc[...]


def _alpha(s, ea, we, att2):
    return pl.pallas_call(
        _alpha_body,
        grid=(NRB,),
        in_specs=[
            pl.BlockSpec((RB, CH), lambda i: (i, 0)),
            pl.BlockSpec((RB, 1), lambda i: (i, 0)),
            pl.BlockSpec((1, CH), lambda i: (0, 0)),
            pl.BlockSpec((CH, 1), lambda i: (0, 0)),
        ],
        out_specs=[
            pl.BlockSpec((RB, 1), lambda i: (i, 0)),
            pl.BlockSpec((1, 1), lambda i: (0, 0)),
        ],
        out_shape=[
            jax.ShapeDtypeStruct((E, 1), jnp.float32),
            jax.ShapeDtypeStruct((1, 1), jnp.float32),
        ],
        scratch_shapes=[pltpu.VMEM((1, 1), jnp.float32)],
    )(s, ea, we, att2)


def _exp_body(a_ref, g_ref, e_ref):
    e_ref[...] = jnp.exp(a_ref[...] - g_ref[0, 0])


def _expk(alpha, gmax):
    return pl.pallas_call(
        _exp_body,
        grid=(NRB,),
        in_specs=[
            pl.BlockSpec((RB, 1), lambda i: (i, 0)),
            pl.BlockSpec((1, 1), lambda i: (0, 0)),
        ],
        out_specs=pl.BlockSpec((RB, 1), lambda i: (i, 0)),
        out_shape=jax.ShapeDtypeStruct((E, 1), jnp.float32),
    )(alpha, gmax)


@functools.partial(
    pl.kernel,
    out_type=jax.ShapeDtypeStruct((NW * N,), jnp.float32),
    mesh=_mesh,
    compiler_params=_sc_params,
    scratch_types=[
        pltpu.VMEM((2 * NCHUNK, B), jnp.int32),
        pltpu.VMEM((N,), jnp.float32),
    ],
)
def _ssum_kernel(comb2_hbm, ssum_p_hbm, comb_a, ssum_v):
    wid = lax.axis_index("s") * NC + lax.axis_index("c")
    pltpu.sync_copy(comb2_hbm.at[wid], comb_a)

    def zbody(i, _):
        ssum_v[pl.ds(i * 16, 16)] = jnp.zeros((16,), jnp.float32)
        return 0

    lax.fori_loop(0, N // 16, zbody, 0)

    def chunk_body(k, _):
        for q in range(B // 16):
            dv = comb_a[k, pl.ds(16 * q, 16)]
            ev = plsc.bitcast(comb_a[NCHUNK + k, pl.ds(16 * q, 16)],
                              jnp.float32)
            plsc.addupdate_scatter(ssum_v, [dv], ev)
        return 0

    lax.fori_loop(0, NCHUNK, chunk_body, 0)
    pltpu.sync_copy(ssum_v, ssum_p_hbm.at[pl.ds(wid * N, N)])


def _rsum_body(p_ref, o_ref):
    o_ref[...] = jnp.sum(p_ref[...], axis=0, keepdims=True)


def _rsum(p):
    return pl.pallas_call(
        _rsum_body,
        out_shape=jax.ShapeDtypeStruct((1, N), jnp.float32),
    )(p)


@functools.partial(
    pl.kernel,
    out_type=jax.ShapeDtypeStruct((NW, NCHUNK, B), jnp.float32),
    mesh=_mesh,
    compiler_params=_sc_params,
    scratch_types=[
        pltpu.VMEM((2 * NCHUNK, B), jnp.int32),
        pltpu.VMEM((NCHUNK, B), jnp.float32),
        pltpu.VMEM((N,), jnp.float32),
    ],
)
def _norm_kernel(comb2_hbm, ssum_hbm, alphan_hbm, comb_a, anv_a, ssum_v):
    wid = lax.axis_index("s") * NC + lax.axis_index("c")
    pltpu.sync_copy(comb2_hbm.at[wid], comb_a)
    pltpu.sync_copy(ssum_hbm, ssum_v)

    def chunk_body(k, _):
        for q in range(B // 16):
            dv = comb_a[k, pl.ds(16 * q, 16)]
            ev = plsc.bitcast(comb_a[NCHUNK + k, pl.ds(16 * q, 16)],
                              jnp.float32)
            sv = plsc.load_gather(ssum_v, [dv])
            anv_a[k, pl.ds(16 * q, 16)] = ev / (sv + 1e-16)
        return 0

    lax.fori_loop(0, NCHUNK, chunk_body, 0)
    pltpu.sync_copy(anv_a, alphan_hbm.at[wid])


@functools.partial(
    pl.kernel,
    out_type=jax.ShapeDtypeStruct((NC, NP, CH), jnp.float32),
    mesh=_mesh,
    compiler_params=_sc_params,
    scratch_types=[
        pltpu.VMEM((NCHUNK, B), jnp.float32),
        pltpu.VMEM((B,), jnp.int32),
        pltpu.VMEM((B,), jnp.int32),
        pltpu.VMEM((B, CH), jnp.float32),
        pltpu.VMEM((WCH, CH), jnp.float32),
        pltpu.VMEM_SHARED((NP, CH), jnp.float32),
        pltpu.SemaphoreType.DMA,
    ],
)
def _scat_kernel(xl_hbm, src_hbm, dst_hbm, ex3_hbm, outp_hbm,
                 exs_a, src_v, dst_v, rows, stage, out_sh, sem):
    cid = lax.axis_index("c")
    sid = lax.axis_index("s")
    wid = sid * NC + cid
    base = wid * EPT
    pltpu.sync_copy(ex3_hbm.at[wid], exs_a)

    def zb(i, _):
        for j in range(8):
            stage[i, pl.ds(16 * j, 16)] = jnp.zeros((16,), jnp.float32)
        return 0

    lax.fori_loop(0, WCH, zb, 0)
    row0 = sid * RPT2

    def zs(t, _):
        pltpu.sync_copy(stage, out_sh.at[pl.ds(row0 + t * WCH, WCH)])
        return 0

    lax.fori_loop(0, RPT2 // WCH, zs, 0)
    plsc.subcore_barrier()

    def chunk_body(k, _):
        off = base + k * B
        pltpu.sync_copy(src_hbm.at[pl.ds(off, B)], src_v)
        pltpu.sync_copy(dst_hbm.at[pl.ds(off, B)], dst_v)
        pltpu.async_copy(xl_hbm.at[src_v], rows, sem).wait()
        for q in range(B // 16):
            av = exs_a[k, pl.ds(16 * q, 16)]
            for i in range(16):
                a = av[i]
                r = 16 * q + i
                for j in range(8):
                    rows[r, pl.ds(16 * j, 16)] = \
                        rows[r, pl.ds(16 * j, 16)] * a
        pltpu.sync_copy(rows, out_sh.at[dst_v], add=True)
        return 0

    lax.fori_loop(0, NCHUNK, chunk_body, 0)
    plsc.subcore_barrier()

    def wb(t, _):
        pltpu.sync_copy(out_sh.at[pl.ds(row0 + t * WCH, WCH)], stage)
        pltpu.sync_copy(stage, outp_hbm.at[cid, pl.ds(row0 + t * WCH, WCH)])
        return 0

    lax.fori_loop(0, RPT2 // WCH, wb, 0)


FB = 1000
NFB = N // FB


def _final_body(p_ref, ssum_ref, bias_ref, linwt_ref, linb_ref, o_ref, acc):
    i = pl.program_id(0)

    @pl.when(i == 0)
    def _():
        acc[...] = jnp.zeros_like(acc)

    o = (p_ref[0] + p_ref[1]) / (ssum_ref[...] + 1e-16) + bias_ref[...]
    h = jnp.maximum(o, 0.0)
    acc[...] += jnp.sum(h, axis=0, keepdims=True)
    pooled = acc[...] * jnp.float32(1.0 / N)
    logits = jnp.dot(pooled, linwt_ref[...], preferred_element_type=jnp.float32) \
        + linb_ref[...]
    m = jnp.max(logits, axis=1, keepdims=True)
    e = jnp.exp(logits - m)
    o_ref[...] = e / jnp.sum(e, axis=1, keepdims=True)


def _final(p, ssum, bias, linwt, linb):
    return pl.pallas_call(
        _final_body,
        grid=(NFB,),
        in_specs=[
            pl.BlockSpec((NC, FB, CH), lambda i: (0, i, 0)),
            pl.BlockSpec((FB, 1), lambda i: (i, 0)),
            pl.BlockSpec((1, CH), lambda i: (0, 0)),
            pl.BlockSpec((CH, 2), lambda i: (0, 0)),
            pl.BlockSpec((1, 2), lambda i: (0, 0)),
        ],
        out_specs=pl.BlockSpec((1, 2), lambda i: (0, 0)),
        out_shape=jax.ShapeDtypeStruct((1, 2), jnp.float32),
        scratch_shapes=[pltpu.VMEM((1, CH), jnp.float32)],
    )(p, ssum, bias, linwt, linb)


@jax.jit
def _run(node_list, edge_list, edge_att, W_l, b_l, W_r, b_r, att, W_e,
         bias, lin_W, lin_b):
    x = node_list[0].astype(jnp.float32)
    src = edge_list[0, 0].astype(jnp.int32)
    dst = edge_list[0, 1].astype(jnp.int32)
    ea = edge_att[0].astype(jnp.float32)
    src3 = src.reshape(NW, NCHUNK, B)
    dst3 = dst.reshape(NW, NCHUNK, B)
    xl, xr = _proj(x, W_l.T, W_r.T, b_l[None, :], b_r[None, :])
    s = _gs_kernel(xl, xr, src3, dst3)
    alpha, gmax = _alpha(s, ea, W_e.T, att[:, None])
    ex = _expk(alpha, gmax)
    ex3i = ex.reshape(NW, NCHUNK, B).view(jnp.int32)
    comb2 = jnp.concatenate([dst3, ex3i], axis=1)

    ssum_p = _ssum_kernel(comb2)
    ssum2 = _rsum(ssum_p.reshape(NW, N))
    alpha_n = _norm_kernel(comb2, ssum2.reshape(-1)).reshape(-1)
    outp = _scat_kernel(xl, src, dst, ex.reshape(NW, NCHUNK, B))
    pred = _final(outp[:, :N], ssum2.reshape(N, 1), bias[None, :], lin_W.T,
                  lin_b[None, :])
    return pred, alpha_n


def kernel(node_list, edge_list, edge_att, W_l, b_l, W_r, b_r, att, W_e,
           bias, lin_W, lin_b):
    return _run(node_list, edge_list, edge_att, W_l, b_l, W_r, b_r, att,
                W_e, bias, lin_W, lin_b)
